# baseline (speedup 1.0000x reference)
import functools
import numpy as np
import jax
import jax.numpy as jnp
from jax.experimental import pallas as pl
from jax.experimental.pallas import tpu as pltpu

NBH = 9
NG = 784
NC = 4096
NT = 1024
CTX_CHUNK = 1024
CHINS = (4, 16, 32, 16)
CHOUTS = (16, 32, 16, 2)
F32 = jnp.float32
_HI = jax.lax.Precision.HIGHEST


_DEF = jax.lax.Precision.DEFAULT


def _dotT(a, b, prec=_DEF):
    return jax.lax.dot_general(a, b, (((1,), (1,)), ((), ())),
                               preferred_element_type=F32, precision=prec)


def _dot0(a, b, prec=_DEF):
    return jax.lax.dot_general(a, b, (((0,), (0,)), ((), ())),
                               preferred_element_type=F32, precision=prec)


def _mm(a, b, prec=_DEF):
    return jax.lax.dot_general(a, b, (((1,), (0,)), ((), ())),
                               preferred_element_type=F32, precision=prec)


def _fwd_body(ctx_ref, val_ref, tgt_ref, tg_ref, tgT_ref, relf_ref, idx_ref,
              scal_ref, bias_ref,
              w1_0, w2_0, w3_0, lin_0,
              w1_1, w2_1, w3_1, lin_1,
              w1_2, w2_2, w3_2, lin_2,
              w1_3, w2_3, w3_3, lin_3,
              mu_ref, sig_ref, wgt_ref):
    wts = ((w1_0, w2_0, w3_0, lin_0), (w1_1, w2_1, w3_1, lin_1),
           (w1_2, w2_2, w3_2, lin_2), (w1_3, w2_3, w3_3, lin_3))

    ls_psi = scal_ref[0, 0]
    os_psi = scal_ref[0, 1]
    ls_rho = scal_ref[0, 2]
    os_rho = scal_ref[0, 3]

    tg = tg_ref[...]
    tgT = tgT_ref[...]
    tg_sq = jnp.sum(tg * tg, axis=1, keepdims=True)

    a_psi = -0.5 / (ls_psi * ls_psi)

    def jbody(j, acc):
        cT = ctx_ref[0, :, pl.ds(j * CTX_CHUNK, CTX_CHUNK)]
        vT = val_ref[0, :, pl.ds(j * CTX_CHUNK, CTX_CHUNK)]
        c_sq = jnp.sum(cT * cT, axis=0, keepdims=True)
        cross = _dot0(tgT, cT, _HI)
        d2 = tg_sq + c_sq - 2.0 * cross
        kp = os_psi * jnp.exp(a_psi * d2)
        vaugT = jnp.concatenate([jnp.ones((1, CTX_CHUNK), F32), vT], axis=0)
        return acc + _dotT(kp, vaugT, _DEF)

    acc = jax.lax.fori_loop(0, NC // CTX_CHUNK, jbody, jnp.zeros((NG, 2), F32))

    h0 = acc[:, 0:1]
    h1 = acc[:, 1:2] / (h0 + 1e-8)
    h = jnp.concatenate([tg, h0, h1], axis=1)

    @pl.when(pl.program_id(0) == 0)
    def _compute_weightnets():
        relf = relf_ref[...]
        for i in range(4):
            w1, w2, w3, _ = wts[i]
            b1 = bias_ref[8 * i + 0:8 * i + 1, 0:32]
            b2 = bias_ref[8 * i + 1:8 * i + 2, 0:64]
            b3 = bias_ref[8 * i + 2:8 * i + 3, 0:16]
            hh = jnp.maximum(_mm(relf, w1[...], _DEF) + b1, 0.0)
            hh = jnp.maximum(_mm(hh, w2[...], _DEF) + b2, 0.0)
            wgt_ref[i * NBH * NG:(i + 1) * NBH * NG, :] = jnp.maximum(
                _mm(hh, w3[...], _DEF) + b3, 0.0)

    row_iota = jax.lax.broadcasted_iota(jnp.int32, (NG, NG), 0)
    inv9 = jnp.float32(1.0 / NBH)

    v_cur = h
    for i in range(4):
        _, _, _, lin = wts[i]
        cin, cout = CHINS[i], CHOUTS[i]
        lb = bias_ref[8 * i + 3:8 * i + 4, 0:cout]

        col = jax.lax.broadcasted_iota(jnp.int32, (1, 16 * cin), 1)
        f2_sel = (col // cin == jax.lax.broadcasted_iota(
            jnp.int32, (16, 16 * cin), 0)).astype(F32)

        vv = v_cur
        base = i * NBH * NG

        def kbody(k, part):
            idxk = idx_ref[pl.ds(k, 1), :]
            s_kt = (row_iota == idxk).astype(F32)
            nb_k = _dot0(s_kt, vv)
            wgt_k = wgt_ref[pl.ds(base + k * NG, NG), :]
            nb_t = jnp.concatenate([nb_k] * 16, axis=1)
            return part + nb_t * _mm(wgt_k, f2_sel)

        part = jax.lax.fori_loop(0, NBH, kbody,
                                 jnp.zeros((NG, 16 * cin), F32))
        out = _mm(part * inv9, lin[...], _DEF) + lb
        v_cur = jnp.maximum(out, 0.0) if i < 3 else out

    f_mu = v_cur[:, 0:1]
    f_sig = jnp.logaddexp(v_cur[:, 1:2], 0.0)

    xT = tgt_ref[0]
    x_sq = _dot0(xT * xT, jnp.ones((2, 1), F32), _HI)
    tg_sq_row = jnp.sum(tgT * tgT, axis=0, keepdims=True)
    cross_r = _dot0(xT, tgT, _HI)
    d2r = x_sq + tg_sq_row - 2.0 * cross_r
    a_rho = -0.5 / (ls_rho * ls_rho)
    kr = os_rho * jnp.exp(a_rho * d2r)
    fstack = jnp.concatenate([f_mu, f_sig], axis=1)
    res = _mm(kr, fstack, _DEF)

    mu_ref[...] = res[:, 0:1][None]
    ri = jax.lax.broadcasted_iota(jnp.int32, (NT, NT), 0)
    ci = jax.lax.broadcasted_iota(jnp.int32, (NT, NT), 1)
    eye = (ri == ci).astype(F32)
    sig_ref[...] = (res[:, 1:2] * eye)[None]


@jax.jit
def _pointcnp_fwd(ctxT, valT, tgtT, params, tg, tgT, relf, idxp, scal, biases):
    B = ctxT.shape[0]
    const = lambda nd: (lambda b: (0,) * nd)
    in_specs = [
        pl.BlockSpec((1, 2, NC), lambda b: (b, 0, 0)),
        pl.BlockSpec((1, 1, NC), lambda b: (b, 0, 0)),
        pl.BlockSpec((1, 2, NT), lambda b: (b, 0, 0)),
        pl.BlockSpec(tg.shape, const(2)),
        pl.BlockSpec(tgT.shape, const(2)),
        pl.BlockSpec(relf.shape, const(2)),
        pl.BlockSpec(idxp.shape, const(2)),
        pl.BlockSpec(scal.shape, const(2)),
        pl.BlockSpec(biases.shape, const(2)),
    ]
    wt_args = []
    for i in range(4):
        w1 = jnp.zeros((8, 32), F32).at[0:2, :].set(params['w%d_1' % i])
        lin_p = params['lin%d_w' % i].reshape(CHINS[i], 16, CHOUTS[i])
        lin_p = lin_p.transpose(1, 0, 2).reshape(16 * CHINS[i], CHOUTS[i])
        for w in (w1, params['w%d_2' % i], params['w%d_3' % i], lin_p):
            wt_args.append(w)
            in_specs.append(pl.BlockSpec(w.shape, const(2)))

    out_shape = [
        jax.ShapeDtypeStruct((B, NT, 1), F32),
        jax.ShapeDtypeStruct((B, NT, NT), F32),
    ]
    out_specs = [
        pl.BlockSpec((1, NT, 1), lambda b: (b, 0, 0)),
        pl.BlockSpec((1, NT, NT), lambda b: (b, 0, 0)),
    ]
    mu3, sigma = pl.pallas_call(
        _fwd_body,
        grid=(B,),
        in_specs=in_specs,
        out_specs=out_specs,
        out_shape=out_shape,
        scratch_shapes=[pltpu.VMEM((4 * NBH * NG, 16), F32)],
    )(ctxT, valT, tgtT, tg, tgT, relf, idxp, scal, biases, *wt_args)
    return mu3[..., 0], sigma


def kernel(ctx_coords, ctx_values, tgt_coords, params):
    i = jnp.linspace(-14.0, 14.0, 28)
    g = jnp.stack(jnp.meshgrid(i, i, indexing='ij'), -1)
    g = g.astype(jnp.float32).reshape(-1, 2)
    d2g = jnp.sum((g[:, None, :] - g[None, :, :]) ** 2, -1)
    idx = jax.lax.top_k(-d2g, NBH)[1]
    rel = g[idx] - g[:, None, :]
    relf = rel.transpose(1, 0, 2).reshape(NBH * NG, 2)
    relf = jnp.concatenate([relf, jnp.zeros((NBH * NG, 6), F32)], axis=1)
    idxp = jnp.zeros((16, NG), jnp.int32).at[:NBH, :].set(idx.T)

    scal = jnp.zeros((8, 128), F32)
    scal = scal.at[0, 0].set(params['ls_psi']).at[0, 1].set(params['os_psi'])
    scal = scal.at[0, 2].set(params['ls_rho']).at[0, 3].set(params['os_rho'])

    biases = jnp.zeros((32, 128), F32)
    for li in range(4):
        biases = biases.at[8 * li + 0, 0:32].set(params['w%d_1b' % li])
        biases = biases.at[8 * li + 1, 0:64].set(params['w%d_2b' % li])
        biases = biases.at[8 * li + 2, 0:16].set(params['w%d_3b' % li])
        biases = biases.at[8 * li + 3, 0:CHOUTS[li]].set(params['lin%d_b' % li])

    ctxT = ctx_coords.transpose(0, 2, 1)
    valT = ctx_values.transpose(0, 2, 1)
    tgtT = tgt_coords.transpose(0, 2, 1)
    return _pointcnp_fwd(ctxT, valT, tgtT, params,
                         g, g.T, relf, idxp, scal, biases)


# speedup vs baseline: 1.2284x; 1.2284x over previous
import functools
import numpy as np
import jax
import jax.numpy as jnp
from jax.experimental import pallas as pl
from jax.experimental.pallas import tpu as pltpu

NBH = 9
NG = 784
NC = 4096
NT = 1024
CTX_CHUNK = 1024
CHINS = (4, 16, 32, 16)
CHOUTS = (16, 32, 16, 2)
F32 = jnp.float32
_HI = jax.lax.Precision.HIGHEST


_DEF = jax.lax.Precision.DEFAULT


def _dotT(a, b, prec=_DEF):
    return jax.lax.dot_general(a, b, (((1,), (1,)), ((), ())),
                               preferred_element_type=F32, precision=prec)


def _dot0(a, b, prec=_DEF):
    return jax.lax.dot_general(a, b, (((0,), (0,)), ((), ())),
                               preferred_element_type=F32, precision=prec)


def _mm(a, b, prec=_DEF):
    return jax.lax.dot_general(a, b, (((1,), (0,)), ((), ())),
                               preferred_element_type=F32, precision=prec)


def _fwd_body(ctx_ref, val_ref, tgt_ref, tg_ref, tgT_ref, relf_ref, idx_ref,
              scal_ref, bias_ref,
              w1_0, w2_0, w3_0, lin_0,
              w1_1, w2_1, w3_1, lin_1,
              w1_2, w2_2, w3_2, lin_2,
              w1_3, w2_3, w3_3, lin_3,
              mu_ref, sig_ref, wgt_ref):
    wts = ((w1_0, w2_0, w3_0, lin_0), (w1_1, w2_1, w3_1, lin_1),
           (w1_2, w2_2, w3_2, lin_2), (w1_3, w2_3, w3_3, lin_3))

    ls_psi = scal_ref[0, 0]
    os_psi = scal_ref[0, 1]
    ls_rho = scal_ref[0, 2]
    os_rho = scal_ref[0, 3]

    tg = tg_ref[...]
    tgT = tgT_ref[...]
    tg_sq = jnp.sum(tg * tg, axis=1, keepdims=True)

    a_psi = -0.5 / (ls_psi * ls_psi)

    def jbody(j, acc):
        cT = ctx_ref[0, :, pl.ds(j * CTX_CHUNK, CTX_CHUNK)]
        vT = val_ref[0, :, pl.ds(j * CTX_CHUNK, CTX_CHUNK)]
        c_sq = jnp.sum(cT * cT, axis=0, keepdims=True)
        cross = _dot0(tgT, cT, _HI)
        d2 = tg_sq + c_sq - 2.0 * cross
        kp = os_psi * jnp.exp(a_psi * d2)
        vaugT = jnp.concatenate([jnp.ones((1, CTX_CHUNK), F32), vT], axis=0)
        return acc + _dotT(kp, vaugT, _DEF)

    acc = jax.lax.fori_loop(0, NC // CTX_CHUNK, jbody, jnp.zeros((NG, 2), F32))

    h0 = acc[:, 0:1]
    h1 = acc[:, 1:2] / (h0 + 1e-8)
    h = jnp.concatenate([tg, h0, h1], axis=1)

    @pl.when(pl.program_id(0) == 0)
    def _compute_weightnets():
        relf = relf_ref[...]
        for i in range(4):
            w1, w2, w3, _ = wts[i]
            b1 = bias_ref[8 * i + 0:8 * i + 1, 0:32]
            b2 = bias_ref[8 * i + 1:8 * i + 2, 0:64]
            b3 = bias_ref[8 * i + 2:8 * i + 3, 0:16]
            hh = jnp.maximum(_mm(relf, w1[...], _DEF) + b1, 0.0)
            hh = jnp.maximum(_mm(hh, w2[...], _DEF) + b2, 0.0)
            wgt_ref[i * NBH * NG:(i + 1) * NBH * NG, :] = jnp.maximum(
                _mm(hh, w3[...], _DEF) + b3, 0.0)

    row_iota = jax.lax.broadcasted_iota(jnp.int32, (NG, NG), 0)
    inv9 = jnp.float32(1.0 / NBH)

    v_cur = h
    for i in range(4):
        _, _, _, lin = wts[i]
        cin, cout = CHINS[i], CHOUTS[i]
        lb = bias_ref[8 * i + 3:8 * i + 4, 0:cout]

        col = jax.lax.broadcasted_iota(jnp.int32, (1, 16 * cin), 1)
        f2_sel = (col // cin == jax.lax.broadcasted_iota(
            jnp.int32, (16, 16 * cin), 0)).astype(F32)
        e2_sel = (col % cin == jax.lax.broadcasted_iota(
            jnp.int32, (cin, 16 * cin), 0)).astype(F32)

        vv = v_cur
        base = i * NBH * NG

        def kbody(k, part):
            idxk = idx_ref[pl.ds(k, 1), :]
            s_kt = (row_iota == idxk).astype(F32)
            nb_k = _dot0(s_kt, vv)
            wgt_k = wgt_ref[pl.ds(base + k * NG, NG), :]
            return part + _mm(nb_k, e2_sel) * _mm(wgt_k, f2_sel)

        part = jax.lax.fori_loop(0, NBH, kbody,
                                 jnp.zeros((NG, 16 * cin), F32))
        out = _mm(part * inv9, lin[...], _DEF) + lb
        v_cur = jnp.maximum(out, 0.0) if i < 3 else out

    f_mu = v_cur[:, 0:1]
    f_sig = jnp.logaddexp(v_cur[:, 1:2], 0.0)

    xT = tgt_ref[0]
    x_sq = _dot0(xT * xT, jnp.ones((2, 1), F32), _HI)
    tg_sq_row = jnp.sum(tgT * tgT, axis=0, keepdims=True)
    cross_r = _dot0(xT, tgT, _HI)
    d2r = x_sq + tg_sq_row - 2.0 * cross_r
    a_rho = -0.5 / (ls_rho * ls_rho)
    kr = os_rho * jnp.exp(a_rho * d2r)
    fstack = jnp.concatenate([f_mu, f_sig], axis=1)
    res = _mm(kr, fstack, _DEF)

    mu_ref[...] = res[:, 0:1][None]
    ri = jax.lax.broadcasted_iota(jnp.int32, (NT, NT), 0)
    ci = jax.lax.broadcasted_iota(jnp.int32, (NT, NT), 1)
    eye = (ri == ci).astype(F32)
    sig_ref[...] = (res[:, 1:2] * eye)[None]


@jax.jit
def _pointcnp_fwd(ctxT, valT, tgtT, params, tg, tgT, relf, idxp, scal, biases):
    B = ctxT.shape[0]
    const = lambda nd: (lambda b: (0,) * nd)
    in_specs = [
        pl.BlockSpec((1, 2, NC), lambda b: (b, 0, 0)),
        pl.BlockSpec((1, 1, NC), lambda b: (b, 0, 0)),
        pl.BlockSpec((1, 2, NT), lambda b: (b, 0, 0)),
        pl.BlockSpec(tg.shape, const(2)),
        pl.BlockSpec(tgT.shape, const(2)),
        pl.BlockSpec(relf.shape, const(2)),
        pl.BlockSpec(idxp.shape, const(2)),
        pl.BlockSpec(scal.shape, const(2)),
        pl.BlockSpec(biases.shape, const(2)),
    ]
    wt_args = []
    for i in range(4):
        w1 = jnp.zeros((8, 32), F32).at[0:2, :].set(params['w%d_1' % i])
        lin_p = params['lin%d_w' % i].reshape(CHINS[i], 16, CHOUTS[i])
        lin_p = lin_p.transpose(1, 0, 2).reshape(16 * CHINS[i], CHOUTS[i])
        for w in (w1, params['w%d_2' % i], params['w%d_3' % i], lin_p):
            wt_args.append(w)
            in_specs.append(pl.BlockSpec(w.shape, const(2)))

    out_shape = [
        jax.ShapeDtypeStruct((B, NT, 1), F32),
        jax.ShapeDtypeStruct((B, NT, NT), F32),
    ]
    out_specs = [
        pl.BlockSpec((1, NT, 1), lambda b: (b, 0, 0)),
        pl.BlockSpec((1, NT, NT), lambda b: (b, 0, 0)),
    ]
    mu3, sigma = pl.pallas_call(
        _fwd_body,
        grid=(B,),
        in_specs=in_specs,
        out_specs=out_specs,
        out_shape=out_shape,
        scratch_shapes=[pltpu.VMEM((4 * NBH * NG, 16), F32)],
    )(ctxT, valT, tgtT, tg, tgT, relf, idxp, scal, biases, *wt_args)
    return mu3[..., 0], sigma


def kernel(ctx_coords, ctx_values, tgt_coords, params):
    i = jnp.linspace(-14.0, 14.0, 28)
    g = jnp.stack(jnp.meshgrid(i, i, indexing='ij'), -1)
    g = g.astype(jnp.float32).reshape(-1, 2)
    d2g = jnp.sum((g[:, None, :] - g[None, :, :]) ** 2, -1)
    idx = jax.lax.top_k(-d2g, NBH)[1]
    rel = g[idx] - g[:, None, :]
    relf = rel.transpose(1, 0, 2).reshape(NBH * NG, 2)
    relf = jnp.concatenate([relf, jnp.zeros((NBH * NG, 6), F32)], axis=1)
    idxp = jnp.zeros((16, NG), jnp.int32).at[:NBH, :].set(idx.T)

    scal = jnp.zeros((8, 128), F32)
    scal = scal.at[0, 0].set(params['ls_psi']).at[0, 1].set(params['os_psi'])
    scal = scal.at[0, 2].set(params['ls_rho']).at[0, 3].set(params['os_rho'])

    biases = jnp.zeros((32, 128), F32)
    for li in range(4):
        biases = biases.at[8 * li + 0, 0:32].set(params['w%d_1b' % li])
        biases = biases.at[8 * li + 1, 0:64].set(params['w%d_2b' % li])
        biases = biases.at[8 * li + 2, 0:16].set(params['w%d_3b' % li])
        biases = biases.at[8 * li + 3, 0:CHOUTS[li]].set(params['lin%d_b' % li])

    ctxT = ctx_coords.transpose(0, 2, 1)
    valT = ctx_values.transpose(0, 2, 1)
    tgtT = tgt_coords.transpose(0, 2, 1)
    return _pointcnp_fwd(ctxT, valT, tgtT, params,
                         g, g.T, relf, idxp, scal, biases)
